# baseline (speedup 1.0000x reference)
import functools

import jax
import jax.numpy as jnp
from jax import lax
from jax.experimental import pallas as pl
from jax.experimental.pallas import tpu as pltpu
from jax.experimental.pallas import tpu_sc as plsc

N_NODES = 10000
N_EDGES = 320000
D_FEAT = 128
DW = D_FEAT // 2

NC = 2
NS = 16
L = 16
NW = NC * NS

C = 256
SUB = 2
SUBLEN = C // SUB
GPC = C // L
NCHUNKS = N_EDGES // C
NIT = 20

_mesh = plsc.VectorSubcoreMesh(core_axis_name="c", subcore_axis_name="s")


@functools.partial(
    pl.kernel,
    mesh=_mesh,
    compiler_params=pltpu.CompilerParams(needs_layout_passes=False,
                                         use_tc_tiling_on_sc=False),
    out_type=jax.ShapeDtypeStruct((N_EDGES,), jnp.float32),
    scratch_types=[
        pltpu.VMEM((SUB, SUBLEN), jnp.int32),
        pltpu.VMEM((SUB, SUBLEN), jnp.int32),
        pltpu.VMEM((SUB, SUBLEN), jnp.int32),
        pltpu.VMEM((SUB, SUBLEN), jnp.int32),
        pltpu.VMEM((C, DW), jnp.int32),
        pltpu.VMEM((C, DW), jnp.int32),
        pltpu.VMEM((C, DW), jnp.int32),
        pltpu.VMEM((C, DW), jnp.int32),
        pltpu.VMEM((C,), jnp.float32),
        pltpu.VMEM((C,), jnp.float32),
        pltpu.SemaphoreType.DMA,
        pltpu.SemaphoreType.DMA,
        pltpu.SemaphoreType.DMA,
        pltpu.SemaphoreType.DMA,
        pltpu.SemaphoreType.DMA,
        pltpu.SemaphoreType.DMA,
    ],
)
def _sc_decode(zi_hbm, ei_hbm, out_hbm,
               col0, row0, col1, row1, av0, bv0, av1, bv1, ov0, ov1,
               isem0, isem1, gsem0, gsem1, osem0, osem1):
    cid = lax.axis_index("c")
    sid = lax.axis_index("s")
    wid = sid * NC + cid
    nw = jnp.where(wid < 2, 40, 39)

    bufs = (
        (col0, row0, av0, bv0, ov0, isem0, gsem0, osem0),
        (col1, row1, av1, bv1, ov1, isem1, gsem1, osem1),
    )

    def ebase(i):
        return (wid + NW * i) * C

    def idx_issue(i, buf):
        colr, rowr, _, _, _, isem, _, _ = buf
        b = ebase(i)
        for sub in range(SUB):
            pltpu.async_copy(
                ei_hbm.at[pl.ds(b + sub * SUBLEN, SUBLEN)], colr.at[sub], isem)
            pltpu.async_copy(
                ei_hbm.at[pl.ds(N_EDGES + b + sub * SUBLEN, SUBLEN)],
                rowr.at[sub], isem)

    def idx_wait(i, buf):
        colr, rowr, _, _, _, isem, _, _ = buf
        b = ebase(i)
        for sub in range(SUB):
            pltpu.make_async_copy(
                ei_hbm.at[pl.ds(b + sub * SUBLEN, SUBLEN)], colr.at[sub],
                isem).wait()
            pltpu.make_async_copy(
                ei_hbm.at[pl.ds(N_EDGES + b + sub * SUBLEN, SUBLEN)],
                rowr.at[sub], isem).wait()

    def gather_issue(buf):
        colr, rowr, avr, bvr, _, _, gsem, _ = buf
        for sub in range(SUB):
            pltpu.async_copy(
                zi_hbm.at[colr.at[sub]], avr.at[pl.ds(sub * SUBLEN, SUBLEN)],
                gsem)
            pltpu.async_copy(
                zi_hbm.at[rowr.at[sub]], bvr.at[pl.ds(sub * SUBLEN, SUBLEN)],
                gsem)

    def gather_wait(buf):
        colr, rowr, avr, bvr, _, _, gsem, _ = buf
        for sub in range(SUB):
            pltpu.make_async_copy(
                zi_hbm.at[colr.at[sub]], avr.at[pl.ds(sub * SUBLEN, SUBLEN)],
                gsem).wait()
            pltpu.make_async_copy(
                zi_hbm.at[rowr.at[sub]], bvr.at[pl.ds(sub * SUBLEN, SUBLEN)],
                gsem).wait()

    def out_issue(i, buf):
        ovr, osem = buf[4], buf[7]
        pltpu.async_copy(ovr, out_hbm.at[pl.ds(ebase(i), C)], osem)

    def out_wait(i, buf):
        ovr, osem = buf[4], buf[7]
        pltpu.make_async_copy(ovr, out_hbm.at[pl.ds(ebase(i), C)], osem).wait()

    def compute(buf):
        avr, bvr, ovr = buf[2], buf[3], buf[4]

        @plsc.parallel_loop(0, GPC, unroll=4)
        def group_body(g):
            lanes = lax.iota(jnp.int32, L)
            eids = g * L + lanes
            accs = [jnp.zeros((L,), jnp.float32) for _ in range(2)]
            for k in range(DW):
                ks = jnp.bitwise_and(lanes + k, DW - 1)
                ga = plsc.load_gather(avr, [eids, ks])
                gb = plsc.load_gather(bvr, [eids, ks])
                prod = plsc.bitcast(ga, jnp.bfloat16) * plsc.bitcast(gb, jnp.bfloat16)
                pe, po = plsc.unpack(prod,
                                     format=plsc.PackFormat.INTERLEAVED,
                                     preferred_element_type=jnp.float32)
                accs[0] = accs[0] + pe
                accs[1] = accs[1] + po
            d = accs[0] + accs[1]
            ovr[pl.ds(g * L, L)] = 1.0 / (1.0 + jnp.exp(-d))

    idx_issue(0, bufs[0])
    idx_issue(1, bufs[1])
    idx_wait(0, bufs[0])
    gather_issue(bufs[0])

    def pipe_body(j, carry):
        for par in (0, 1):
            i = 2 * j + par
            buf = bufs[par]
            nxt = bufs[1 - par]

            def chunk_step():
                @pl.when(i + 1 < nw)
                def _():
                    idx_wait(i + 1, nxt)
                    gather_issue(nxt)

                gather_wait(buf)

                @pl.when(i + 2 < nw)
                def _():
                    idx_issue(i + 2, buf)

                @pl.when(i >= 2)
                def _():
                    out_wait(i - 2, buf)

                compute(buf)
                out_issue(i, buf)

            if par == 0:
                chunk_step()
            else:
                pl.when(i < nw)(chunk_step)
        return carry

    lax.fori_loop(0, NIT, pipe_body, 0)

    out_wait(nw - 2 + ((nw - 2) % 2 == 1), bufs[0])
    out_wait(nw - 2 + ((nw - 2) % 2 == 0), bufs[1])


def kernel(z, edge_index):
    zb = z.astype(jnp.bfloat16)
    zi = lax.bitcast_convert_type(zb.reshape(N_NODES, DW, 2), jnp.int32)
    return _sc_decode(zi, edge_index.reshape(-1))


# speedup vs baseline: 1.0535x; 1.0535x over previous
import functools

import jax
import jax.numpy as jnp
from jax import lax
from jax.experimental import pallas as pl
from jax.experimental.pallas import tpu as pltpu
from jax.experimental.pallas import tpu_sc as plsc

N_NODES = 10000
N_EDGES = 320000
D_FEAT = 128
DW = D_FEAT // 2

NC = 2
NS = 16
L = 16
NW = NC * NS

C = 256
SUB = 2
SUBLEN = C // SUB
GPC = C // L
NCHUNKS = N_EDGES // C
NIT = 20

_mesh = plsc.VectorSubcoreMesh(core_axis_name="c", subcore_axis_name="s")


@functools.partial(
    pl.kernel,
    mesh=_mesh,
    compiler_params=pltpu.CompilerParams(needs_layout_passes=False,
                                         use_tc_tiling_on_sc=False),
    out_type=jax.ShapeDtypeStruct((N_EDGES,), jnp.float32),
    scratch_types=[
        pltpu.VMEM((SUB, SUBLEN), jnp.int32),
        pltpu.VMEM((SUB, SUBLEN), jnp.int32),
        pltpu.VMEM((SUB, SUBLEN), jnp.int32),
        pltpu.VMEM((SUB, SUBLEN), jnp.int32),
        pltpu.VMEM((C, DW), jnp.int32),
        pltpu.VMEM((C, DW), jnp.int32),
        pltpu.VMEM((C, DW), jnp.int32),
        pltpu.VMEM((C, DW), jnp.int32),
        pltpu.VMEM((C,), jnp.float32),
        pltpu.VMEM((C,), jnp.float32),
        pltpu.SemaphoreType.DMA,
        pltpu.SemaphoreType.DMA,
        pltpu.SemaphoreType.DMA,
        pltpu.SemaphoreType.DMA,
        pltpu.SemaphoreType.DMA,
        pltpu.SemaphoreType.DMA,
    ],
)
def _sc_decode(zi_hbm, ei_hbm, out_hbm,
               col0, row0, col1, row1, av0, bv0, av1, bv1, ov0, ov1,
               isem0, isem1, gsem0, gsem1, osem0, osem1):
    cid = lax.axis_index("c")
    sid = lax.axis_index("s")
    wid = sid * NC + cid
    nw = jnp.where(wid < 2, 40, 39)

    bufs = (
        (col0, row0, av0, bv0, ov0, isem0, gsem0, osem0),
        (col1, row1, av1, bv1, ov1, isem1, gsem1, osem1),
    )

    def ebase(i):
        return (wid + NW * i) * C

    def idx_issue(i, buf):
        colr, rowr, _, _, _, isem, _, _ = buf
        b = ebase(i)
        for sub in range(SUB):
            pltpu.async_copy(
                ei_hbm.at[pl.ds(b + sub * SUBLEN, SUBLEN)], colr.at[sub], isem)
            pltpu.async_copy(
                ei_hbm.at[pl.ds(N_EDGES + b + sub * SUBLEN, SUBLEN)],
                rowr.at[sub], isem)

    def idx_wait(i, buf):
        colr, rowr, _, _, _, isem, _, _ = buf
        b = ebase(i)
        for sub in range(SUB):
            pltpu.make_async_copy(
                ei_hbm.at[pl.ds(b + sub * SUBLEN, SUBLEN)], colr.at[sub],
                isem).wait()
            pltpu.make_async_copy(
                ei_hbm.at[pl.ds(N_EDGES + b + sub * SUBLEN, SUBLEN)],
                rowr.at[sub], isem).wait()

    def gather_issue(buf):
        colr, rowr, avr, bvr, _, _, gsem, _ = buf
        for sub in range(SUB):
            pltpu.async_copy(
                zi_hbm.at[colr.at[sub]], avr.at[pl.ds(sub * SUBLEN, SUBLEN)],
                gsem)
            pltpu.async_copy(
                zi_hbm.at[rowr.at[sub]], bvr.at[pl.ds(sub * SUBLEN, SUBLEN)],
                gsem)

    def gather_wait(buf):
        colr, rowr, avr, bvr, _, _, gsem, _ = buf
        for sub in range(SUB):
            pltpu.make_async_copy(
                zi_hbm.at[colr.at[sub]], avr.at[pl.ds(sub * SUBLEN, SUBLEN)],
                gsem).wait()
            pltpu.make_async_copy(
                zi_hbm.at[rowr.at[sub]], bvr.at[pl.ds(sub * SUBLEN, SUBLEN)],
                gsem).wait()

    def out_issue(i, buf):
        ovr, osem = buf[4], buf[7]
        pltpu.async_copy(ovr, out_hbm.at[pl.ds(ebase(i), C)], osem)

    def out_wait(i, buf):
        ovr, osem = buf[4], buf[7]
        pltpu.make_async_copy(ovr, out_hbm.at[pl.ds(ebase(i), C)], osem).wait()

    def compute(buf):
        avr, bvr, ovr = buf[2], buf[3], buf[4]

        @plsc.parallel_loop(0, GPC, unroll=2)
        def group_body(g):
            lanes = lax.iota(jnp.int32, L)
            eids = g * L + lanes
            accs = [jnp.zeros((L,), jnp.float32) for _ in range(2)]
            ks = lanes
            for k in range(DW):
                if k:
                    ks = jnp.bitwise_and(ks + 1, DW - 1)
                ga = plsc.load_gather(avr, [eids, ks])
                gb = plsc.load_gather(bvr, [eids, ks])
                prod = plsc.bitcast(ga, jnp.bfloat16) * plsc.bitcast(gb, jnp.bfloat16)
                pe, po = plsc.unpack(prod,
                                     format=plsc.PackFormat.INTERLEAVED,
                                     preferred_element_type=jnp.float32)
                accs[0] = accs[0] + pe
                accs[1] = accs[1] + po
            d = accs[0] + accs[1]
            ovr[pl.ds(g * L, L)] = 1.0 / (1.0 + jnp.exp(-d))

    idx_issue(0, bufs[0])
    idx_issue(1, bufs[1])
    idx_wait(0, bufs[0])
    gather_issue(bufs[0])

    def pipe_body(j, carry):
        for par in (0, 1):
            i = 2 * j + par
            buf = bufs[par]
            nxt = bufs[1 - par]

            def chunk_step():
                @pl.when(i + 1 < nw)
                def _():
                    idx_wait(i + 1, nxt)
                    gather_issue(nxt)

                gather_wait(buf)

                @pl.when(i + 2 < nw)
                def _():
                    idx_issue(i + 2, buf)

                @pl.when(i >= 2)
                def _():
                    out_wait(i - 2, buf)

                compute(buf)
                out_issue(i, buf)

            if par == 0:
                chunk_step()
            else:
                pl.when(i < nw)(chunk_step)
        return carry

    lax.fori_loop(0, NIT, pipe_body, 0)

    out_wait(nw - 2 + ((nw - 2) % 2 == 1), bufs[0])
    out_wait(nw - 2 + ((nw - 2) % 2 == 0), bufs[1])


def kernel(z, edge_index):
    zb = z.astype(jnp.bfloat16)
    zi = lax.bitcast_convert_type(zb.reshape(N_NODES, DW, 2), jnp.int32)
    return _sc_decode(zi, edge_index.reshape(-1))


# speedup vs baseline: 1.0671x; 1.0129x over previous
import functools

import jax
import jax.numpy as jnp
from jax import lax
from jax.experimental import pallas as pl
from jax.experimental.pallas import tpu as pltpu
from jax.experimental.pallas import tpu_sc as plsc

N_NODES = 10000
N_EDGES = 320000
D_FEAT = 128
DW = D_FEAT // 2

NC = 2
NS = 16
L = 16
NW = NC * NS

C = 256
SUB = 2
SUBLEN = C // SUB
GPC = C // L
NCHUNKS = N_EDGES // C
NIT = 20

_mesh = plsc.VectorSubcoreMesh(core_axis_name="c", subcore_axis_name="s")


@functools.partial(
    pl.kernel,
    mesh=_mesh,
    compiler_params=pltpu.CompilerParams(needs_layout_passes=False,
                                         use_tc_tiling_on_sc=False),
    out_type=jax.ShapeDtypeStruct((N_EDGES,), jnp.float32),
    scratch_types=[
        pltpu.VMEM((SUB, SUBLEN), jnp.int32),
        pltpu.VMEM((SUB, SUBLEN), jnp.int32),
        pltpu.VMEM((SUB, SUBLEN), jnp.int32),
        pltpu.VMEM((SUB, SUBLEN), jnp.int32),
        pltpu.VMEM((C, DW), jnp.int32),
        pltpu.VMEM((C, DW), jnp.int32),
        pltpu.VMEM((C, DW), jnp.int32),
        pltpu.VMEM((C, DW), jnp.int32),
        pltpu.VMEM((C,), jnp.float32),
        pltpu.VMEM((C,), jnp.float32),
        pltpu.VMEM_SHARED((N_NODES, DW), jnp.int32),
        pltpu.SemaphoreType.DMA,
        pltpu.SemaphoreType.DMA,
        pltpu.SemaphoreType.DMA,
        pltpu.SemaphoreType.DMA,
        pltpu.SemaphoreType.DMA,
        pltpu.SemaphoreType.DMA,
    ],
)
def _sc_decode(zi_hbm, ei_hbm, out_hbm,
               col0, row0, col1, row1, av0, bv0, av1, bv1, ov0, ov1, zsp,
               isem0, isem1, gsem0, gsem1, osem0, osem1):
    cid = lax.axis_index("c")
    sid = lax.axis_index("s")
    wid = sid * NC + cid
    nw = jnp.where(wid < 2, 40, 39)

    bufs = (
        (col0, row0, av0, bv0, ov0, isem0, gsem0, osem0),
        (col1, row1, av1, bv1, ov1, isem1, gsem1, osem1),
    )

    def ebase(i):
        return (wid + NW * i) * C

    def idx_issue(i, buf):
        colr, rowr, _, _, _, isem, _, _ = buf
        b = ebase(i)
        for sub in range(SUB):
            pltpu.async_copy(
                ei_hbm.at[pl.ds(b + sub * SUBLEN, SUBLEN)], colr.at[sub], isem)
            pltpu.async_copy(
                ei_hbm.at[pl.ds(N_EDGES + b + sub * SUBLEN, SUBLEN)],
                rowr.at[sub], isem)

    def idx_wait(i, buf):
        colr, rowr, _, _, _, isem, _, _ = buf
        b = ebase(i)
        for sub in range(SUB):
            pltpu.make_async_copy(
                ei_hbm.at[pl.ds(b + sub * SUBLEN, SUBLEN)], colr.at[sub],
                isem).wait()
            pltpu.make_async_copy(
                ei_hbm.at[pl.ds(N_EDGES + b + sub * SUBLEN, SUBLEN)],
                rowr.at[sub], isem).wait()

    def gather_issue(buf):
        colr, rowr, avr, bvr, _, _, gsem, _ = buf
        for sub in range(SUB):
            pltpu.async_copy(
                zsp.at[colr.at[sub]], avr.at[pl.ds(sub * SUBLEN, SUBLEN)],
                gsem)
            pltpu.async_copy(
                zsp.at[rowr.at[sub]], bvr.at[pl.ds(sub * SUBLEN, SUBLEN)],
                gsem)

    def gather_wait(buf):
        colr, rowr, avr, bvr, _, _, gsem, _ = buf
        for sub in range(SUB):
            pltpu.make_async_copy(
                zsp.at[colr.at[sub]], avr.at[pl.ds(sub * SUBLEN, SUBLEN)],
                gsem).wait()
            pltpu.make_async_copy(
                zsp.at[rowr.at[sub]], bvr.at[pl.ds(sub * SUBLEN, SUBLEN)],
                gsem).wait()

    def out_issue(i, buf):
        ovr, osem = buf[4], buf[7]
        pltpu.async_copy(ovr, out_hbm.at[pl.ds(ebase(i), C)], osem)

    def out_wait(i, buf):
        ovr, osem = buf[4], buf[7]
        pltpu.make_async_copy(ovr, out_hbm.at[pl.ds(ebase(i), C)], osem).wait()

    def compute(buf):
        avr, bvr, ovr = buf[2], buf[3], buf[4]

        @plsc.parallel_loop(0, GPC, unroll=2)
        def group_body(g):
            lanes = lax.iota(jnp.int32, L)
            eids = g * L + lanes
            accs = [jnp.zeros((L,), jnp.float32) for _ in range(2)]
            ks = lanes
            for k in range(DW):
                if k:
                    ks = jnp.bitwise_and(ks + 1, DW - 1)
                ga = plsc.load_gather(avr, [eids, ks])
                gb = plsc.load_gather(bvr, [eids, ks])
                prod = plsc.bitcast(ga, jnp.bfloat16) * plsc.bitcast(gb, jnp.bfloat16)
                pe, po = plsc.unpack(prod,
                                     format=plsc.PackFormat.INTERLEAVED,
                                     preferred_element_type=jnp.float32)
                accs[0] = accs[0] + pe
                accs[1] = accs[1] + po
            d = accs[0] + accs[1]
            ovr[pl.ds(g * L, L)] = 1.0 / (1.0 + jnp.exp(-d))

    idx_issue(0, bufs[0])
    idx_issue(1, bufs[1])
    rpt = N_NODES // NS
    pltpu.sync_copy(zi_hbm.at[pl.ds(sid * rpt, rpt)],
                    zsp.at[pl.ds(sid * rpt, rpt)])
    plsc.subcore_barrier()
    idx_wait(0, bufs[0])
    gather_issue(bufs[0])

    def pipe_body(j, carry):
        for par in (0, 1):
            i = 2 * j + par
            buf = bufs[par]
            nxt = bufs[1 - par]

            def chunk_step():
                @pl.when(i + 1 < nw)
                def _():
                    idx_wait(i + 1, nxt)
                    gather_issue(nxt)

                gather_wait(buf)

                @pl.when(i + 2 < nw)
                def _():
                    idx_issue(i + 2, buf)

                @pl.when(i >= 2)
                def _():
                    out_wait(i - 2, buf)

                compute(buf)
                out_issue(i, buf)

            if par == 0:
                chunk_step()
            else:
                pl.when(i < nw)(chunk_step)
        return carry

    lax.fori_loop(0, NIT, pipe_body, 0)

    out_wait(nw - 2 + ((nw - 2) % 2 == 1), bufs[0])
    out_wait(nw - 2 + ((nw - 2) % 2 == 0), bufs[1])


def kernel(z, edge_index):
    zb = z.astype(jnp.bfloat16)
    zi = lax.bitcast_convert_type(zb.reshape(N_NODES, DW, 2), jnp.int32)
    return _sc_decode(zi, edge_index.reshape(-1))


# speedup vs baseline: 1.0672x; 1.0000x over previous
import functools

import jax
import jax.numpy as jnp
from jax import lax
from jax.experimental import pallas as pl
from jax.experimental.pallas import tpu as pltpu
from jax.experimental.pallas import tpu_sc as plsc

N_NODES = 10000
N_EDGES = 320000
D_FEAT = 128
DW = D_FEAT // 2

NC = 2
NS = 16
L = 16
NW = NC * NS

C = 256
SUB = 1
SUBLEN = C // SUB
GPC = C // L
NCHUNKS = N_EDGES // C
NIT = 20

_mesh = plsc.VectorSubcoreMesh(core_axis_name="c", subcore_axis_name="s")


@functools.partial(
    pl.kernel,
    mesh=_mesh,
    compiler_params=pltpu.CompilerParams(needs_layout_passes=False,
                                         use_tc_tiling_on_sc=False),
    out_type=jax.ShapeDtypeStruct((N_EDGES,), jnp.float32),
    scratch_types=[
        pltpu.VMEM((SUB, SUBLEN), jnp.int32),
        pltpu.VMEM((SUB, SUBLEN), jnp.int32),
        pltpu.VMEM((SUB, SUBLEN), jnp.int32),
        pltpu.VMEM((SUB, SUBLEN), jnp.int32),
        pltpu.VMEM((C, DW), jnp.int32),
        pltpu.VMEM((C, DW), jnp.int32),
        pltpu.VMEM((C, DW), jnp.int32),
        pltpu.VMEM((C, DW), jnp.int32),
        pltpu.VMEM((C,), jnp.float32),
        pltpu.VMEM((C,), jnp.float32),
        pltpu.VMEM_SHARED((N_NODES, DW), jnp.int32),
        pltpu.SemaphoreType.DMA,
        pltpu.SemaphoreType.DMA,
        pltpu.SemaphoreType.DMA,
        pltpu.SemaphoreType.DMA,
        pltpu.SemaphoreType.DMA,
        pltpu.SemaphoreType.DMA,
    ],
)
def _sc_decode(zi_hbm, ei_hbm, out_hbm,
               col0, row0, col1, row1, av0, bv0, av1, bv1, ov0, ov1, zsp,
               isem0, isem1, gsem0, gsem1, osem0, osem1):
    cid = lax.axis_index("c")
    sid = lax.axis_index("s")
    wid = sid * NC + cid
    nw = jnp.where(wid < 2, 40, 39)

    bufs = (
        (col0, row0, av0, bv0, ov0, isem0, gsem0, osem0),
        (col1, row1, av1, bv1, ov1, isem1, gsem1, osem1),
    )

    def ebase(i):
        return (wid + NW * i) * C

    def idx_issue(i, buf):
        colr, rowr, _, _, _, isem, _, _ = buf
        b = ebase(i)
        for sub in range(SUB):
            pltpu.async_copy(
                ei_hbm.at[pl.ds(b + sub * SUBLEN, SUBLEN)], colr.at[sub], isem)
            pltpu.async_copy(
                ei_hbm.at[pl.ds(N_EDGES + b + sub * SUBLEN, SUBLEN)],
                rowr.at[sub], isem)

    def idx_wait(i, buf):
        colr, rowr, _, _, _, isem, _, _ = buf
        b = ebase(i)
        for sub in range(SUB):
            pltpu.make_async_copy(
                ei_hbm.at[pl.ds(b + sub * SUBLEN, SUBLEN)], colr.at[sub],
                isem).wait()
            pltpu.make_async_copy(
                ei_hbm.at[pl.ds(N_EDGES + b + sub * SUBLEN, SUBLEN)],
                rowr.at[sub], isem).wait()

    def gather_issue(buf):
        colr, rowr, avr, bvr, _, _, gsem, _ = buf
        for sub in range(SUB):
            pltpu.async_copy(
                zsp.at[colr.at[sub]], avr.at[pl.ds(sub * SUBLEN, SUBLEN)],
                gsem)
            pltpu.async_copy(
                zsp.at[rowr.at[sub]], bvr.at[pl.ds(sub * SUBLEN, SUBLEN)],
                gsem)

    def gather_wait(buf):
        colr, rowr, avr, bvr, _, _, gsem, _ = buf
        for sub in range(SUB):
            pltpu.make_async_copy(
                zsp.at[colr.at[sub]], avr.at[pl.ds(sub * SUBLEN, SUBLEN)],
                gsem).wait()
            pltpu.make_async_copy(
                zsp.at[rowr.at[sub]], bvr.at[pl.ds(sub * SUBLEN, SUBLEN)],
                gsem).wait()

    def out_issue(i, buf):
        ovr, osem = buf[4], buf[7]
        pltpu.async_copy(ovr, out_hbm.at[pl.ds(ebase(i), C)], osem)

    def out_wait(i, buf):
        ovr, osem = buf[4], buf[7]
        pltpu.make_async_copy(ovr, out_hbm.at[pl.ds(ebase(i), C)], osem).wait()

    def compute(buf):
        avr, bvr, ovr = buf[2], buf[3], buf[4]

        @plsc.parallel_loop(0, GPC, unroll=2)
        def group_body(g):
            lanes = lax.iota(jnp.int32, L)
            eids = g * L + lanes
            accs = [jnp.zeros((L,), jnp.float32) for _ in range(2)]
            ks = lanes
            for k in range(DW):
                if k:
                    ks = jnp.bitwise_and(ks + 1, DW - 1)
                ga = plsc.load_gather(avr, [eids, ks])
                gb = plsc.load_gather(bvr, [eids, ks])
                prod = plsc.bitcast(ga, jnp.bfloat16) * plsc.bitcast(gb, jnp.bfloat16)
                pe, po = plsc.unpack(prod,
                                     format=plsc.PackFormat.INTERLEAVED,
                                     preferred_element_type=jnp.float32)
                accs[0] = accs[0] + pe
                accs[1] = accs[1] + po
            d = accs[0] + accs[1]
            ovr[pl.ds(g * L, L)] = 1.0 / (1.0 + jnp.exp(-d))

    idx_issue(0, bufs[0])
    idx_issue(1, bufs[1])
    rpt = N_NODES // NS
    pltpu.sync_copy(zi_hbm.at[pl.ds(sid * rpt, rpt)],
                    zsp.at[pl.ds(sid * rpt, rpt)])
    plsc.subcore_barrier()
    idx_wait(0, bufs[0])
    gather_issue(bufs[0])

    def pipe_body(j, carry):
        for par in (0, 1):
            i = 2 * j + par
            buf = bufs[par]
            nxt = bufs[1 - par]

            def chunk_step():
                @pl.when(i + 1 < nw)
                def _():
                    idx_wait(i + 1, nxt)
                    gather_issue(nxt)

                gather_wait(buf)

                @pl.when(i + 2 < nw)
                def _():
                    idx_issue(i + 2, buf)

                @pl.when(i >= 2)
                def _():
                    out_wait(i - 2, buf)

                compute(buf)
                out_issue(i, buf)

            if par == 0:
                chunk_step()
            else:
                pl.when(i < nw)(chunk_step)
        return carry

    lax.fori_loop(0, NIT, pipe_body, 0)

    out_wait(nw - 2 + ((nw - 2) % 2 == 1), bufs[0])
    out_wait(nw - 2 + ((nw - 2) % 2 == 0), bufs[1])


def kernel(z, edge_index):
    zb = z.astype(jnp.bfloat16)
    zi = lax.bitcast_convert_type(zb.reshape(N_NODES, DW, 2), jnp.int32)
    return _sc_decode(zi, edge_index.reshape(-1))


# speedup vs baseline: 1.0766x; 1.0088x over previous
import functools

import jax
import jax.numpy as jnp
from jax import lax
from jax.experimental import pallas as pl
from jax.experimental.pallas import tpu as pltpu
from jax.experimental.pallas import tpu_sc as plsc

N_NODES = 10000
N_EDGES = 320000
D_FEAT = 128
DW = D_FEAT // 2

NC = 2
NS = 16
L = 16
NW = NC * NS

C = 400
SUB = 1
SUBLEN = C // SUB
GPC = C // L
NCHUNKS = N_EDGES // C
NIT = 13

_mesh = plsc.VectorSubcoreMesh(core_axis_name="c", subcore_axis_name="s")


@functools.partial(
    pl.kernel,
    mesh=_mesh,
    compiler_params=pltpu.CompilerParams(needs_layout_passes=False,
                                         use_tc_tiling_on_sc=False),
    out_type=jax.ShapeDtypeStruct((N_EDGES,), jnp.float32),
    scratch_types=[
        pltpu.VMEM((SUB, SUBLEN), jnp.int32),
        pltpu.VMEM((SUB, SUBLEN), jnp.int32),
        pltpu.VMEM((SUB, SUBLEN), jnp.int32),
        pltpu.VMEM((SUB, SUBLEN), jnp.int32),
        pltpu.VMEM((C, DW), jnp.int32),
        pltpu.VMEM((C, DW), jnp.int32),
        pltpu.VMEM((C, DW), jnp.int32),
        pltpu.VMEM((C, DW), jnp.int32),
        pltpu.VMEM((C,), jnp.float32),
        pltpu.VMEM((C,), jnp.float32),
        pltpu.SemaphoreType.DMA,
        pltpu.SemaphoreType.DMA,
        pltpu.SemaphoreType.DMA,
        pltpu.SemaphoreType.DMA,
        pltpu.SemaphoreType.DMA,
        pltpu.SemaphoreType.DMA,
    ],
)
def _sc_decode(zi_hbm, ei_hbm, out_hbm,
               col0, row0, col1, row1, av0, bv0, av1, bv1, ov0, ov1,
               isem0, isem1, gsem0, gsem1, osem0, osem1):
    cid = lax.axis_index("c")
    sid = lax.axis_index("s")
    wid = sid * NC + cid
    nw = jnp.int32(NCHUNKS // NW)

    bufs = (
        (col0, row0, av0, bv0, ov0, isem0, gsem0, osem0),
        (col1, row1, av1, bv1, ov1, isem1, gsem1, osem1),
    )

    def ebase(i):
        return (wid + NW * i) * C

    def idx_issue(i, buf):
        colr, rowr, _, _, _, isem, _, _ = buf
        b = ebase(i)
        for sub in range(SUB):
            pltpu.async_copy(
                ei_hbm.at[pl.ds(b + sub * SUBLEN, SUBLEN)], colr.at[sub], isem)
            pltpu.async_copy(
                ei_hbm.at[pl.ds(N_EDGES + b + sub * SUBLEN, SUBLEN)],
                rowr.at[sub], isem)

    def idx_wait(i, buf):
        colr, rowr, _, _, _, isem, _, _ = buf
        b = ebase(i)
        for sub in range(SUB):
            pltpu.make_async_copy(
                ei_hbm.at[pl.ds(b + sub * SUBLEN, SUBLEN)], colr.at[sub],
                isem).wait()
            pltpu.make_async_copy(
                ei_hbm.at[pl.ds(N_EDGES + b + sub * SUBLEN, SUBLEN)],
                rowr.at[sub], isem).wait()

    def gather_issue(buf):
        colr, rowr, avr, bvr, _, _, gsem, _ = buf
        for sub in range(SUB):
            pltpu.async_copy(
                zi_hbm.at[colr.at[sub]], avr.at[pl.ds(sub * SUBLEN, SUBLEN)],
                gsem)
            pltpu.async_copy(
                zi_hbm.at[rowr.at[sub]], bvr.at[pl.ds(sub * SUBLEN, SUBLEN)],
                gsem)

    def gather_wait(buf):
        colr, rowr, avr, bvr, _, _, gsem, _ = buf
        for sub in range(SUB):
            pltpu.make_async_copy(
                zi_hbm.at[colr.at[sub]], avr.at[pl.ds(sub * SUBLEN, SUBLEN)],
                gsem).wait()
            pltpu.make_async_copy(
                zi_hbm.at[rowr.at[sub]], bvr.at[pl.ds(sub * SUBLEN, SUBLEN)],
                gsem).wait()

    def out_issue(i, buf):
        ovr, osem = buf[4], buf[7]
        pltpu.async_copy(ovr, out_hbm.at[pl.ds(ebase(i), C)], osem)

    def out_wait(i, buf):
        ovr, osem = buf[4], buf[7]
        pltpu.make_async_copy(ovr, out_hbm.at[pl.ds(ebase(i), C)], osem).wait()

    def compute(buf):
        avr, bvr, ovr = buf[2], buf[3], buf[4]

        @plsc.parallel_loop(0, GPC, unroll=2)
        def group_body(g):
            lanes = lax.iota(jnp.int32, L)
            eids = g * L + lanes
            accs = [jnp.zeros((L,), jnp.float32) for _ in range(2)]
            ks = lanes
            for k in range(DW):
                if k:
                    ks = jnp.bitwise_and(ks + 1, DW - 1)
                ga = plsc.load_gather(avr, [eids, ks])
                gb = plsc.load_gather(bvr, [eids, ks])
                prod = plsc.bitcast(ga, jnp.bfloat16) * plsc.bitcast(gb, jnp.bfloat16)
                pe, po = plsc.unpack(prod,
                                     format=plsc.PackFormat.INTERLEAVED,
                                     preferred_element_type=jnp.float32)
                accs[0] = accs[0] + pe
                accs[1] = accs[1] + po
            d = accs[0] + accs[1]
            ovr[pl.ds(g * L, L)] = 1.0 / (1.0 + jnp.exp(-d))

    idx_issue(0, bufs[0])
    idx_issue(1, bufs[1])
    idx_wait(0, bufs[0])
    gather_issue(bufs[0])

    def pipe_body(j, carry):
        for par in (0, 1):
            i = 2 * j + par
            buf = bufs[par]
            nxt = bufs[1 - par]

            def chunk_step():
                @pl.when(i + 1 < nw)
                def _():
                    idx_wait(i + 1, nxt)
                    gather_issue(nxt)

                gather_wait(buf)

                @pl.when(i + 2 < nw)
                def _():
                    idx_issue(i + 2, buf)

                @pl.when(i >= 2)
                def _():
                    out_wait(i - 2, buf)

                compute(buf)
                out_issue(i, buf)

            if par == 0:
                chunk_step()
            else:
                pl.when(i < nw)(chunk_step)
        return carry

    lax.fori_loop(0, NIT, pipe_body, 0)

    out_wait(nw - 2 + ((nw - 2) % 2 == 1), bufs[0])
    out_wait(nw - 2 + ((nw - 2) % 2 == 0), bufs[1])


def kernel(z, edge_index):
    zb = z.astype(jnp.bfloat16)
    zi = lax.bitcast_convert_type(zb.reshape(N_NODES, DW, 2), jnp.int32)
    return _sc_decode(zi, edge_index.reshape(-1))
